# baseline (speedup 1.0000x reference)
import functools

import jax
import jax.numpy as jnp
from jax.experimental import pallas as pl

_ANCHORS = 3
_HEIGHT = 256
_WIDTH = 256
_OUT_P = 1000
_IOU_THR = 0.5
_RATIO = 0.3
_SHAPES = [(128, 128), (64, 64), (32, 32)]
_B = 4
_C = 64
_TOTAL = sum(h * w for h, w in _SHAPES) * _ANCHORS
_TOPK = int(_TOTAL * _RATIO)


def _conv3x3(x, w, b):
    out = jax.lax.conv_general_dilated(x, w, (1, 1), 'SAME',
                                       dimension_numbers=('NHWC', 'HWIO', 'NHWC'))
    return out + b


def _conv1x1(x, w, b):
    return jnp.einsum('bhwc,co->bhwo', x, w[0, 0]) + b


def _centers(H, W):
    rows = jnp.arange(0, _HEIGHT, _HEIGHT // H, dtype=jnp.float32)
    cols = jnp.arange(0, _WIDTH, _WIDTH // W, dtype=jnp.float32)
    cx = jnp.tile(cols[None, :], (H, 1))
    cy = jnp.tile(rows[:, None], (1, W))
    grid = jnp.stack([cx, cy], axis=-1)
    return jnp.repeat(grid.reshape(H * W, 1, 2), _ANCHORS, axis=1).reshape(1, H * W * _ANCHORS, 2)


def _sizes(H, W):
    return jnp.array([[[_HEIGHT / H, _WIDTH / W]]], dtype=jnp.float32)


def _sig_kernel(x_ref, out_ref):
    out_ref[...] = jax.nn.sigmoid(x_ref[...])


def _sig(x):
    return pl.pallas_call(
        _sig_kernel,
        out_shape=jax.ShapeDtypeStruct(x.shape, jnp.float32),
    )(x)


def _decode(bbox, centers, sizes):
    img = jnp.array([[[float(_HEIGHT), float(_WIDTH)]]], dtype=jnp.float32)
    xy = bbox[..., :2] * img + centers
    hw = (jnp.exp(bbox[..., 2:]) * img + sizes) / 2.0
    lo = jnp.clip(xy - hw, jnp.array([0.0, 0.0]), jnp.array([float(_HEIGHT), float(_WIDTH)]))
    hi = jnp.clip(xy + hw, jnp.array([0.0, 0.0]), jnp.array([float(_HEIGHT), float(_WIDTH)]))
    return jnp.concatenate([lo, hi], axis=-1)


def _nms_pad(scores, boxes):
    scores_c = jax.lax.stop_gradient(scores)
    boxes_c = jax.lax.stop_gradient(boxes)
    N = scores_c.shape[0]
    neg = jnp.float32(-1e30)
    areas = jnp.maximum(boxes_c[:, 2] - boxes_c[:, 0], 0.0) * jnp.maximum(boxes_c[:, 3] - boxes_c[:, 1], 0.0)

    def body(valid, _):
        ms = jnp.where(valid, scores_c, neg)
        best = jnp.argmax(ms)
        best_ok = ms[best] > neg * 0.5
        bb = boxes_c[best]
        yy1 = jnp.maximum(bb[0], boxes_c[:, 0])
        xx1 = jnp.maximum(bb[1], boxes_c[:, 1])
        yy2 = jnp.minimum(bb[2], boxes_c[:, 2])
        xx2 = jnp.minimum(bb[3], boxes_c[:, 3])
        inter = jnp.maximum(yy2 - yy1, 0.0) * jnp.maximum(xx2 - xx1, 0.0)
        union = areas[best] + areas - inter
        iou = jnp.where(union > 0.0, inter / union, 0.0)
        new_valid = valid & (iou <= _IOU_THR) & best_ok
        new_valid = new_valid.at[best].set(False)
        return new_valid, (jnp.where(best_ok, best, 0).astype(jnp.int32), best_ok)

    valid0 = jnp.ones((N,), dtype=bool)
    _, (idxs, oks) = jax.lax.scan(body, valid0, None, length=_OUT_P)
    n_kept = jnp.sum(oks.astype(jnp.int32))
    additional = _OUT_P - n_kept
    last_idx = jnp.max(jnp.where(oks, idxs, 0))
    starting = jnp.minimum(_TOPK - additional, last_idx + 1)
    j = jnp.arange(_OUT_P, dtype=jnp.int32)
    fi = jnp.where(j < n_kept, idxs, starting + (j - n_kept))
    fi = jnp.clip(fi, 0, N - 1)
    return scores[fi], boxes[fi]


def kernel(feat0, feat1, feat2, W_in0, b_in0, W_bb0, b_bb0, W_cf0, b_cf0,
           W_in1, b_in1, W_bb1, b_bb1, W_cf1, b_cf1,
           W_in2, b_in2, W_bb2, b_bb2, W_cf2, b_cf2):
    feats = [feat0, feat1, feat2]
    Wi = [W_in0, W_in1, W_in2]
    bi = [b_in0, b_in1, b_in2]
    Wb = [W_bb0, W_bb1, W_bb2]
    bb = [b_bb0, b_bb1, b_bb2]
    Wc = [W_cf0, W_cf1, W_cf2]
    bc = [b_cf0, b_cf1, b_cf2]
    confs = []
    boxes = []
    for s, (H, W) in enumerate(_SHAPES):
        f = jax.nn.relu(_conv3x3(feats[s], Wi[s], bi[s]))
        c = _sig(_conv1x1(f, Wc[s], bc[s]).reshape(_B, H * W * _ANCHORS))
        bx = _conv1x1(f, Wb[s], bb[s]).reshape(_B, H * W * _ANCHORS, 4)
        bx = _decode(bx, _centers(H, W), _sizes(H, W))
        confs.append(c)
        boxes.append(bx)
    conf = jnp.concatenate(confs, axis=-1)
    bbox = jnp.concatenate(boxes, axis=1)
    vals, idx = jax.lax.top_k(conf, _TOPK)
    bbox_k = jnp.take_along_axis(bbox, idx[..., None], axis=1)
    conf_out, box_out = jax.vmap(_nms_pad)(vals, bbox_k)
    return conf_out, box_out


# speedup vs baseline: 10.5657x; 10.5657x over previous
import functools

import jax
import jax.numpy as jnp
from jax.experimental import pallas as pl

_ANCHORS = 3
_HEIGHT = 256
_WIDTH = 256
_OUT_P = 1000
_IOU_THR = 0.5
_RATIO = 0.3
_SHAPES = [(128, 128), (64, 64), (32, 32)]
_B = 4
_C = 64
_TOTAL = sum(h * w for h, w in _SHAPES) * _ANCHORS
_TOPK = int(_TOTAL * _RATIO)


def _conv3x3(x, w, b):
    out = jax.lax.conv_general_dilated(x, w, (1, 1), 'SAME',
                                       dimension_numbers=('NHWC', 'HWIO', 'NHWC'))
    return out + b


def _conv1x1(x, w, b):
    return jnp.einsum('bhwc,co->bhwo', x, w[0, 0]) + b


def _centers(H, W):
    rows = jnp.arange(0, _HEIGHT, _HEIGHT // H, dtype=jnp.float32)
    cols = jnp.arange(0, _WIDTH, _WIDTH // W, dtype=jnp.float32)
    cx = jnp.tile(cols[None, :], (H, 1))
    cy = jnp.tile(rows[:, None], (1, W))
    grid = jnp.stack([cx, cy], axis=-1)
    return jnp.repeat(grid.reshape(H * W, 1, 2), _ANCHORS, axis=1).reshape(1, H * W * _ANCHORS, 2)


def _sizes(H, W):
    return jnp.array([[[_HEIGHT / H, _WIDTH / W]]], dtype=jnp.float32)


def _sig_kernel(x_ref, out_ref):
    out_ref[...] = jax.nn.sigmoid(x_ref[...])


def _sig(x):
    return pl.pallas_call(
        _sig_kernel,
        out_shape=jax.ShapeDtypeStruct(x.shape, jnp.float32),
    )(x)


def _decode(bbox, centers, sizes):
    img = jnp.array([[[float(_HEIGHT), float(_WIDTH)]]], dtype=jnp.float32)
    xy = bbox[..., :2] * img + centers
    hw = (jnp.exp(bbox[..., 2:]) * img + sizes) / 2.0
    lo = jnp.clip(xy - hw, jnp.array([0.0, 0.0]), jnp.array([float(_HEIGHT), float(_WIDTH)]))
    hi = jnp.clip(xy + hw, jnp.array([0.0, 0.0]), jnp.array([float(_HEIGHT), float(_WIDTH)]))
    return jnp.concatenate([lo, hi], axis=-1)


_ROWS = 152
_BIG = 2**30


def _nms_kernel(boxes_ref, fi_ref, valid_ref, kept_ref):
    c0 = boxes_ref[0, 0]
    c1 = boxes_ref[0, 1]
    c2 = boxes_ref[0, 2]
    c3 = boxes_ref[0, 3]
    area = jnp.maximum(c2 - c0, 0.0) * jnp.maximum(c3 - c1, 0.0)
    ridx = jax.lax.broadcasted_iota(jnp.int32, (_ROWS, 128), 0)
    cidx = jax.lax.broadcasted_iota(jnp.int32, (_ROWS, 128), 1)
    flat = ridx * 128 + cidx
    valid_ref[...] = (flat < _TOPK).astype(jnp.float32)
    kept_ref[...] = jnp.zeros((8, 128), jnp.int32)
    oj = jax.lax.broadcasted_iota(jnp.int32, (8, 128), 0) * 128 + \
        jax.lax.broadcasted_iota(jnp.int32, (8, 128), 1)

    def cond(carry):
        fv, n_kept, last = carry
        return (fv < _BIG) & (n_kept < _OUT_P)

    def body(carry):
        fv, n_kept, last = carry
        onehot = flat == fv
        neg = jnp.float32(-1e30)
        b0 = jnp.max(jnp.where(onehot, c0, neg))
        b1 = jnp.max(jnp.where(onehot, c1, neg))
        b2 = jnp.max(jnp.where(onehot, c2, neg))
        b3 = jnp.max(jnp.where(onehot, c3, neg))
        area_b = jnp.maximum(b2 - b0, 0.0) * jnp.maximum(b3 - b1, 0.0)
        yy1 = jnp.maximum(b0, c0)
        xx1 = jnp.maximum(b1, c1)
        yy2 = jnp.minimum(b2, c2)
        xx2 = jnp.minimum(b3, c3)
        inter = jnp.maximum(yy2 - yy1, 0.0) * jnp.maximum(xx2 - xx1, 0.0)
        union = area_b + area - inter
        iou = jnp.where(union > 0.0, inter / union, 0.0)
        v = valid_ref[...] > 0.5
        v = v & (iou <= _IOU_THR) & jnp.logical_not(onehot)
        valid_ref[...] = v.astype(jnp.float32)
        kept_ref[...] = kept_ref[...] + jnp.where(oj == n_kept, fv, 0)
        fv_new = jnp.min(jnp.where(v, flat, _BIG))
        return fv_new, n_kept + 1, fv

    _, n_kept, last = jax.lax.while_loop(cond, body, (jnp.int32(0), jnp.int32(0), jnp.int32(0)))

    additional = _OUT_P - n_kept
    starting = jnp.minimum(_TOPK - additional, last + 1)
    fi = jnp.where(oj < n_kept, kept_ref[...], starting + (oj - n_kept))
    fi_ref[0] = jnp.clip(fi, 0, _TOPK - 1)


def _nms_fi(boxes_t):
    from jax.experimental.pallas import tpu as pltpu
    B = boxes_t.shape[0]
    fi = pl.pallas_call(
        _nms_kernel,
        grid=(B,),
        in_specs=[pl.BlockSpec((1, 4, _ROWS, 128), lambda b: (b, 0, 0, 0))],
        out_specs=pl.BlockSpec((1, 8, 128), lambda b: (b, 0, 0)),
        out_shape=jax.ShapeDtypeStruct((B, 8, 128), jnp.int32),
        scratch_shapes=[pltpu.VMEM((_ROWS, 128), jnp.float32),
                        pltpu.VMEM((8, 128), jnp.int32)],
    )(boxes_t)
    return fi.reshape(B, 1024)


def kernel(feat0, feat1, feat2, W_in0, b_in0, W_bb0, b_bb0, W_cf0, b_cf0,
           W_in1, b_in1, W_bb1, b_bb1, W_cf1, b_cf1,
           W_in2, b_in2, W_bb2, b_bb2, W_cf2, b_cf2):
    feats = [feat0, feat1, feat2]
    Wi = [W_in0, W_in1, W_in2]
    bi = [b_in0, b_in1, b_in2]
    Wb = [W_bb0, W_bb1, W_bb2]
    bb = [b_bb0, b_bb1, b_bb2]
    Wc = [W_cf0, W_cf1, W_cf2]
    bc = [b_cf0, b_cf1, b_cf2]
    confs = []
    boxes = []
    for s, (H, W) in enumerate(_SHAPES):
        f = jax.nn.relu(_conv3x3(feats[s], Wi[s], bi[s]))
        c = _sig(_conv1x1(f, Wc[s], bc[s]).reshape(_B, H * W * _ANCHORS))
        bx = _conv1x1(f, Wb[s], bb[s]).reshape(_B, H * W * _ANCHORS, 4)
        bx = _decode(bx, _centers(H, W), _sizes(H, W))
        confs.append(c)
        boxes.append(bx)
    conf = jnp.concatenate(confs, axis=-1)
    bbox = jnp.concatenate(boxes, axis=1)
    vals, idx = jax.lax.top_k(conf, _TOPK)
    bbox_k = jnp.take_along_axis(bbox, idx[..., None], axis=1)
    bt = jnp.moveaxis(bbox_k, -1, 1)
    bt = jnp.pad(bt, ((0, 0), (0, 0), (0, _ROWS * 128 - _TOPK)))
    bt = bt.reshape(_B, 4, _ROWS, 128)
    fi = _nms_fi(bt)[:, :_OUT_P]
    conf_out = jnp.take_along_axis(vals, fi, axis=1)
    box_out = jnp.take_along_axis(bbox_k, fi[..., None], axis=1)
    return conf_out, box_out
